# Initial kernel scaffold; baseline (speedup 1.0000x reference)
import jax
import jax.numpy as jnp
from jax.experimental import pallas as pl


def kernel(x, edge_index, W, att_src, att_dst, bias):
    raise NotImplementedError("write your pallas kernel here")


# speedup vs baseline: 21.9551x; 21.9551x over previous
import functools

import jax
import jax.numpy as jnp
from jax import lax
from jax.experimental import pallas as pl
from jax.experimental.pallas import tpu as pltpu
from jax.experimental.pallas import tpu_sc as plsc

N = 100000
D = 32
H = D // 2
E = 1600000
NP = 100096
NPS = NP // 16

EP = 1638400
EPR = EP // 128

A_ROWS = EPR // 32
A_CH = 16
A_NCH = A_ROWS // A_CH

B_ROWS = EPR // 16
B_CH = 8
B_NCH = B_ROWS // B_CH
B_STG = 368

BN = 4352
G1 = NP // BN

_mesh = lambda: plsc.VectorSubcoreMesh(
    core_axis_name="c", subcore_axis_name="s", num_cores=2, num_subcores=16)


def _tc1_body(x_ref, w_ref, sel_ref, hA_ref, hB_ref, as_ref, ad_ref,
              ms_ref, md_ref):
    i = pl.program_id(0)
    h = jnp.dot(x_ref[...], w_ref[...], preferred_element_type=jnp.float32)
    hA_ref[...] = h[:, :H]
    hB_ref[...] = h[:, H:]
    a8 = lax.dot_general(sel_ref[...], h, (((1,), (1,)), ((), ())),
                         preferred_element_type=jnp.float32)
    as_ref[...] = a8[0:1, :]
    ad_ref[...] = a8[1:2, :]
    bs = jnp.max(a8[0:1, :])
    bd = jnp.max(a8[1:2, :])

    @pl.when(i == 0)
    def _():
        ms_ref[0, 0] = bs
        md_ref[0, 0] = bd

    @pl.when(i > 0)
    def _():
        ms_ref[0, 0] = jnp.maximum(ms_ref[0, 0], bs)
        md_ref[0, 0] = jnp.maximum(md_ref[0, 0], bd)


def _tc1(xp, W, sel8):
    return pl.pallas_call(
        _tc1_body,
        grid=(G1,),
        in_specs=[
            pl.BlockSpec((BN, D), lambda i: (i, 0)),
            pl.BlockSpec((D, D), lambda i: (0, 0)),
            pl.BlockSpec((8, D), lambda i: (0, 0)),
        ],
        out_specs=[
            pl.BlockSpec((BN, H), lambda i: (i, 0)),
            pl.BlockSpec((BN, H), lambda i: (i, 0)),
            pl.BlockSpec((1, BN), lambda i: (0, i)),
            pl.BlockSpec((1, BN), lambda i: (0, i)),
            pl.BlockSpec((1, 1), lambda i: (0, 0), memory_space=pltpu.SMEM),
            pl.BlockSpec((1, 1), lambda i: (0, 0), memory_space=pltpu.SMEM),
        ],
        out_shape=[
            jax.ShapeDtypeStruct((NP, H), jnp.float32),
            jax.ShapeDtypeStruct((NP, H), jnp.float32),
            jax.ShapeDtypeStruct((1, NP), jnp.float32),
            jax.ShapeDtypeStruct((1, NP), jnp.float32),
            jax.ShapeDtypeStruct((1, 1), jnp.float32),
            jax.ShapeDtypeStruct((1, 1), jnp.float32),
        ],
    )(xp, W, sel8)


@functools.partial(
    pl.kernel,
    out_type=jax.ShapeDtypeStruct((EPR, 128), jnp.float32),
    mesh=_mesh(),
    compiler_params=pltpu.CompilerParams(needs_layout_passes=False, use_tc_tiling_on_sc=False),
    scratch_types=[
        pltpu.VMEM((NP,), jnp.float32),
        pltpu.VMEM((A_CH, 128), jnp.int32),
        pltpu.VMEM((A_CH, 128), jnp.float32),
    ],
)
def _sc_a1(asrc_hbm, src_hbm, p_hbm, tab_v, idx_v, p_v):
    c = lax.axis_index("c")
    s = lax.axis_index("s")
    wid = s * 2 + c
    pltpu.sync_copy(asrc_hbm, tab_v)

    def chunk(t, carry):
        rowbase = wid * A_ROWS + t * A_CH
        pltpu.sync_copy(src_hbm.at[pl.ds(rowbase, A_CH)], idx_v)

        def vec(j, carry2):
            for l in range(8):
                i16 = idx_v[j, pl.ds(l * 16, 16)]
                p_v[j, pl.ds(l * 16, 16)] = plsc.load_gather(tab_v, [i16])
            return carry2

        lax.fori_loop(0, A_CH, vec, 0)
        pltpu.sync_copy(p_v, p_hbm.at[pl.ds(rowbase, A_CH)])
        return carry

    lax.fori_loop(0, A_NCH, chunk, 0)


@functools.partial(
    pl.kernel,
    out_type=[
        jax.ShapeDtypeStruct((EPR, 128), jnp.float32),
        jax.ShapeDtypeStruct((2 * NP,), jnp.float32),
    ],
    mesh=_mesh(),
    compiler_params=pltpu.CompilerParams(needs_layout_passes=False, use_tc_tiling_on_sc=False),
    scratch_types=[
        pltpu.VMEM((NP,), jnp.float32),
        pltpu.VMEM((A_CH, 128), jnp.int32),
        pltpu.VMEM((A_CH, 128), jnp.float32),
        pltpu.VMEM((A_CH, 128), jnp.float32),
        pltpu.VMEM((16,), jnp.float32),
        pltpu.VMEM((4352,), jnp.float32),
        pltpu.VMEM_SHARED((NP,), jnp.float32),
    ],
)
def _sc_a2(adst_hbm, dst_hbm, p_hbm, m_hbm, w_hbm, den_hbm,
           tab_v, idx_v, p_v, w_v, m_v, stage_v, den_sp):
    c = lax.axis_index("c")
    s = lax.axis_index("s")
    wid = s * 2 + c
    z16 = jnp.zeros((16,), jnp.float32)

    @pl.when(s == 0)
    def _():
        def zero(r, carry):
            stage_v[pl.ds(r * 16, 16)] = z16
            return carry

        lax.fori_loop(0, 4352 // 16, zero, 0)

        def zcp(k, carry):
            pltpu.sync_copy(stage_v, den_sp.at[pl.ds(k * 4352, 4352)])
            return carry

        lax.fori_loop(0, NP // 4352, zcp, 0)

    pltpu.sync_copy(adst_hbm, tab_v)
    pltpu.sync_copy(m_hbm, m_v)
    plsc.subcore_barrier()
    mv = m_v[...]

    def chunk(t, carry):
        rowbase = wid * A_ROWS + t * A_CH
        pltpu.sync_copy(dst_hbm.at[pl.ds(rowbase, A_CH)], idx_v)
        pltpu.sync_copy(p_hbm.at[pl.ds(rowbase, A_CH)], p_v)

        def vec(j, carry2):
            for l in range(8):
                i16 = idx_v[j, pl.ds(l * 16, 16)]
                q = plsc.load_gather(tab_v, [i16])
                z = p_v[j, pl.ds(l * 16, 16)] + q
                lr = jnp.where(z >= 0, z, 0.2 * z)
                w_v[j, pl.ds(l * 16, 16)] = jnp.exp(lr - mv)
            return carry2

        lax.fori_loop(0, A_CH, vec, 0)
        pltpu.sync_copy(w_v, w_hbm.at[pl.ds(rowbase, A_CH)])

        def scat(j, carry2):
            pltpu.sync_copy(w_v.at[j], den_sp.at[idx_v.at[j]], add=True)
            return carry2

        lax.fori_loop(0, A_CH, scat, 0)
        return carry

    lax.fori_loop(0, A_NCH, chunk, 0)
    plsc.subcore_barrier()

    @pl.when(s == 0)
    def _():
        def wb(k, carry):
            pltpu.sync_copy(den_sp.at[pl.ds(k * 4352, 4352)], stage_v)
            pltpu.sync_copy(stage_v,
                            den_hbm.at[pl.ds(c * NP + k * 4352, 4352)])
            return carry

        lax.fori_loop(0, NP // 4352, wb, 0)


@functools.partial(
    pl.kernel,
    out_type=jax.ShapeDtypeStruct((2 * NP, H), jnp.float32),
    mesh=_mesh(),
    compiler_params=pltpu.CompilerParams(needs_layout_passes=False, use_tc_tiling_on_sc=False),
    scratch_types=[
        pltpu.VMEM((B_CH, 128), jnp.int32),
        pltpu.VMEM((B_CH, 128), jnp.int32),
        pltpu.VMEM((B_CH, 128), jnp.float32),
        pltpu.VMEM((B_CH * 128, H), jnp.float32),
        pltpu.VMEM((B_STG, H), jnp.float32),
        pltpu.VMEM_SHARED((NP, H), jnp.float32),
        pltpu.SemaphoreType.DMA,
    ],
)
def _sc_b(hA_hbm, hB_hbm, src_hbm, dst_hbm, w_hbm, num_hbm,
          src_v, dst_v, w_v, rows_v, stage_v, num_sp, sem):
    c = lax.axis_index("c")
    s = lax.axis_index("s")
    z16 = jnp.zeros((16,), jnp.float32)

    def zero(r, carry):
        stage_v[r, :] = z16
        return carry

    lax.fori_loop(0, B_STG, zero, 0)

    def zcp(k, carry):
        pltpu.sync_copy(stage_v, num_sp.at[pl.ds(s * NPS + k * B_STG, B_STG)])
        return carry

    lax.fori_loop(0, NPS // B_STG, zcp, 0)
    plsc.subcore_barrier()
    li = lax.iota(jnp.int32, 16)

    def chunk(t, carry):
        rowbase = s * B_ROWS + t * B_CH
        pltpu.sync_copy(src_hbm.at[pl.ds(rowbase, B_CH)], src_v)
        pltpu.sync_copy(dst_hbm.at[pl.ds(rowbase, B_CH)], dst_v)
        pltpu.sync_copy(w_hbm.at[pl.ds(rowbase, B_CH)], w_v)

        def gather(j, carry2):
            @pl.when(c == 0)
            def _():
                pltpu.async_copy(hA_hbm.at[src_v.at[j]],
                                 rows_v.at[pl.ds(j * 128, 128)], sem).wait()

            @pl.when(c == 1)
            def _():
                pltpu.async_copy(hB_hbm.at[src_v.at[j]],
                                 rows_v.at[pl.ds(j * 128, 128)], sem).wait()

            return carry2

        lax.fori_loop(0, B_CH, gather, 0)

        def scale(j, carry2):
            for l in range(8):
                w16 = w_v[j, pl.ds(l * 16, 16)]
                ri = (j * 128 + l * 16) + li
                for f in range(H):
                    cf = jnp.full((16,), f, jnp.int32)
                    g = plsc.load_gather(rows_v, [ri, cf])
                    plsc.store_scatter(rows_v, [ri, cf], g * w16)
            return carry2

        lax.fori_loop(0, B_CH, scale, 0)

        def scat(j, carry2):
            pltpu.sync_copy(rows_v.at[pl.ds(j * 128, 128)],
                            num_sp.at[dst_v.at[j]], add=True)
            return carry2

        lax.fori_loop(0, B_CH, scat, 0)
        return carry

    lax.fori_loop(0, B_NCH, chunk, 0)
    plsc.subcore_barrier()
    def wb(k, carry):
        off = s * NPS + k * B_STG
        pltpu.sync_copy(num_sp.at[pl.ds(off, B_STG)], stage_v)
        pltpu.sync_copy(stage_v, num_hbm.at[pl.ds(c * NP + off, B_STG)])
        return carry

    lax.fori_loop(0, NPS // B_STG, wb, 0)


def _tc2_body(hA_ref, hB_ref, as_ref, ad_ref, d0_ref, d1_ref, n0_ref, n1_ref,
              ms_ref, md_ref, b_ref, o_ref):
    a_s = as_ref[...]
    a_d = ad_ref[...]
    M = jnp.maximum(ms_ref[0, 0] + md_ref[0, 0], 0.0)
    z = a_s + a_d
    lr = jnp.where(z >= 0, z, 0.2 * z)
    wself = jnp.exp(lr - M)
    den = d0_ref[...] + d1_ref[...] + wself + 1e-16
    nA = (n0_ref[...] + wself * hA_ref[...]) / den
    nB = (n1_ref[...] + wself * hB_ref[...]) / den
    o_ref[...] = jnp.concatenate([nA, nB], axis=1) + b_ref[0:1, :]


def _tc2(hA, hB, as_col, ad_col, d0, d1, n0, n1, ms, md, bias8):
    col = pl.BlockSpec((BN, 1), lambda i: (i, 0))
    half = pl.BlockSpec((BN, H), lambda i: (i, 0))
    return pl.pallas_call(
        _tc2_body,
        grid=(G1,),
        in_specs=[
            half, half, col, col, col, col, half, half,
            pl.BlockSpec((1, 1), lambda i: (0, 0), memory_space=pltpu.SMEM),
            pl.BlockSpec((1, 1), lambda i: (0, 0), memory_space=pltpu.SMEM),
            pl.BlockSpec((8, D), lambda i: (0, 0)),
        ],
        out_specs=pl.BlockSpec((BN, D), lambda i: (i, 0)),
        out_shape=jax.ShapeDtypeStruct((NP, D), jnp.float32),
    )(hA, hB, as_col, ad_col, d0, d1, n0, n1, ms, md, bias8)


def kernel(x, edge_index, W, att_src, att_dst, bias):
    src = edge_index[0].astype(jnp.int32)
    dst = edge_index[1].astype(jnp.int32)

    xp = jnp.concatenate([x.astype(jnp.float32),
                          jnp.zeros((NP - N, D), jnp.float32)], axis=0)
    sel8 = jnp.concatenate([att_src[None], att_dst[None],
                            jnp.zeros((6, D), jnp.float32)], axis=0)
    bias8 = jnp.broadcast_to(bias[None, :], (8, D)).astype(jnp.float32)

    hA, hB, asH, adH, ms, md = _tc1(xp, W.astype(jnp.float32), sel8)

    pad = EP - E
    srcP = jnp.concatenate([src, jnp.zeros((pad,), jnp.int32)]).reshape(EPR, 128)
    dstP = jnp.concatenate([dst, jnp.full((pad,), N, jnp.int32)]).reshape(EPR, 128)

    M16 = jnp.broadcast_to(jnp.maximum(ms[0, 0] + md[0, 0], 0.0), (16,))

    p_edge = _sc_a1(asH.reshape(NP), srcP)
    w_edge, den = _sc_a2(adH.reshape(NP), dstP, p_edge, M16)
    num = _sc_b(hA, hB, srcP, dstP, w_edge)

    as_col = asH.reshape(NP, 1)
    ad_col = adH.reshape(NP, 1)
    d0 = den[:NP].reshape(NP, 1)
    d1 = den[NP:].reshape(NP, 1)
    n0 = num[:NP]
    n1 = num[NP:]

    out = _tc2(hA, hB, as_col, ad_col, d0, d1, n0, n1, ms, md, bias8)
    return out[:N]


# speedup vs baseline: 27.0991x; 1.2343x over previous
import functools

import jax
import jax.numpy as jnp
from jax import lax
from jax.experimental import pallas as pl
from jax.experimental.pallas import tpu as pltpu
from jax.experimental.pallas import tpu_sc as plsc

N = 100000
D = 32
H = D // 2
E = 1600000
NP = 100096
NPS = NP // 16

EP = 1638400
EPR = EP // 128

A_ROWS = EPR // 32
A_CH = 16
A_NCH = A_ROWS // A_CH

B_ROWS = EPR // 16
B_CH = 8
B_NCH = B_ROWS // B_CH
B_STG = 368

BN = 4352
G1 = NP // BN

_mesh = lambda: plsc.VectorSubcoreMesh(
    core_axis_name="c", subcore_axis_name="s", num_cores=2, num_subcores=16)


def _tc1_body(x_ref, w_ref, sel_ref, hA_ref, hB_ref, as_ref, ad_ref,
              ms_ref, md_ref):
    i = pl.program_id(0)
    h = jnp.dot(x_ref[...], w_ref[...], preferred_element_type=jnp.float32)
    hA_ref[...] = h[:, :H]
    hB_ref[...] = h[:, H:]
    a8 = lax.dot_general(sel_ref[...], h, (((1,), (1,)), ((), ())),
                         preferred_element_type=jnp.float32)
    as_ref[...] = a8[0:1, :]
    ad_ref[...] = a8[1:2, :]
    bs = jnp.max(a8[0:1, :])
    bd = jnp.max(a8[1:2, :])

    @pl.when(i == 0)
    def _():
        ms_ref[0, 0] = bs
        md_ref[0, 0] = bd

    @pl.when(i > 0)
    def _():
        ms_ref[0, 0] = jnp.maximum(ms_ref[0, 0], bs)
        md_ref[0, 0] = jnp.maximum(md_ref[0, 0], bd)


def _tc1(xp, W, sel8):
    return pl.pallas_call(
        _tc1_body,
        grid=(G1,),
        in_specs=[
            pl.BlockSpec((BN, D), lambda i: (i, 0)),
            pl.BlockSpec((D, D), lambda i: (0, 0)),
            pl.BlockSpec((8, D), lambda i: (0, 0)),
        ],
        out_specs=[
            pl.BlockSpec((BN, H), lambda i: (i, 0)),
            pl.BlockSpec((BN, H), lambda i: (i, 0)),
            pl.BlockSpec((1, BN), lambda i: (0, i)),
            pl.BlockSpec((1, BN), lambda i: (0, i)),
            pl.BlockSpec((1, 1), lambda i: (0, 0), memory_space=pltpu.SMEM),
            pl.BlockSpec((1, 1), lambda i: (0, 0), memory_space=pltpu.SMEM),
        ],
        out_shape=[
            jax.ShapeDtypeStruct((NP, H), jnp.float32),
            jax.ShapeDtypeStruct((NP, H), jnp.float32),
            jax.ShapeDtypeStruct((1, NP), jnp.float32),
            jax.ShapeDtypeStruct((1, NP), jnp.float32),
            jax.ShapeDtypeStruct((1, 1), jnp.float32),
            jax.ShapeDtypeStruct((1, 1), jnp.float32),
        ],
    )(xp, W, sel8)


@functools.partial(
    pl.kernel,
    out_type=jax.ShapeDtypeStruct((EPR, 128), jnp.float32),
    mesh=_mesh(),
    compiler_params=pltpu.CompilerParams(needs_layout_passes=False, use_tc_tiling_on_sc=False),
    scratch_types=[
        pltpu.VMEM((NP,), jnp.float32),
        pltpu.VMEM((A_CH, 128), jnp.int32),
        pltpu.VMEM((A_CH, 128), jnp.float32),
    ],
)
def _sc_a1(asrc_hbm, src_hbm, p_hbm, tab_v, idx_v, p_v):
    c = lax.axis_index("c")
    s = lax.axis_index("s")
    wid = s * 2 + c
    pltpu.sync_copy(asrc_hbm, tab_v)

    def chunk(t, carry):
        rowbase = wid * A_ROWS + t * A_CH
        pltpu.sync_copy(src_hbm.at[pl.ds(rowbase, A_CH)], idx_v)

        def vec(j, carry2):
            for l in range(8):
                i16 = idx_v[j, pl.ds(l * 16, 16)]
                p_v[j, pl.ds(l * 16, 16)] = plsc.load_gather(tab_v, [i16])
            return carry2

        lax.fori_loop(0, A_CH, vec, 0)
        pltpu.sync_copy(p_v, p_hbm.at[pl.ds(rowbase, A_CH)])
        return carry

    lax.fori_loop(0, A_NCH, chunk, 0)


@functools.partial(
    pl.kernel,
    out_type=[
        jax.ShapeDtypeStruct((EPR, 128), jnp.float32),
        jax.ShapeDtypeStruct((2 * NP,), jnp.float32),
    ],
    mesh=_mesh(),
    compiler_params=pltpu.CompilerParams(needs_layout_passes=False, use_tc_tiling_on_sc=False),
    scratch_types=[
        pltpu.VMEM((NP,), jnp.float32),
        pltpu.VMEM((A_CH, 128), jnp.int32),
        pltpu.VMEM((A_CH, 128), jnp.float32),
        pltpu.VMEM((A_CH, 128), jnp.float32),
        pltpu.VMEM((16,), jnp.float32),
        pltpu.VMEM((4352,), jnp.float32),
        pltpu.VMEM_SHARED((NP,), jnp.float32),
        pltpu.SemaphoreType.DMA,
    ],
)
def _sc_a2(adst_hbm, dst_hbm, p_hbm, m_hbm, w_hbm, den_hbm,
           tab_v, idx_v, p_v, w_v, m_v, stage_v, den_sp, sem_a):
    c = lax.axis_index("c")
    s = lax.axis_index("s")
    wid = s * 2 + c
    z16 = jnp.zeros((16,), jnp.float32)

    @pl.when(s == 0)
    def _():
        def zero(r, carry):
            stage_v[pl.ds(r * 16, 16)] = z16
            return carry

        lax.fori_loop(0, 4352 // 16, zero, 0)

        def zcp(k, carry):
            pltpu.sync_copy(stage_v, den_sp.at[pl.ds(k * 4352, 4352)])
            return carry

        lax.fori_loop(0, NP // 4352, zcp, 0)

    pltpu.sync_copy(adst_hbm, tab_v)
    pltpu.sync_copy(m_hbm, m_v)
    plsc.subcore_barrier()
    mv = m_v[...]

    def chunk(t, carry):
        rowbase = wid * A_ROWS + t * A_CH
        pltpu.sync_copy(dst_hbm.at[pl.ds(rowbase, A_CH)], idx_v)
        pltpu.sync_copy(p_hbm.at[pl.ds(rowbase, A_CH)], p_v)

        def vec(j, carry2):
            for l in range(8):
                i16 = idx_v[j, pl.ds(l * 16, 16)]
                q = plsc.load_gather(tab_v, [i16])
                z = p_v[j, pl.ds(l * 16, 16)] + q
                lr = jnp.where(z >= 0, z, 0.2 * z)
                w_v[j, pl.ds(l * 16, 16)] = jnp.exp(lr - mv)
            return carry2

        lax.fori_loop(0, A_CH, vec, 0)
        pltpu.sync_copy(w_v, w_hbm.at[pl.ds(rowbase, A_CH)])

        cps = [pltpu.async_copy(w_v.at[j], den_sp.at[idx_v.at[j]],
                                sem_a, add=True)
               for j in range(A_CH)]
        for cp in cps:
            cp.wait()
        return carry

    lax.fori_loop(0, A_NCH, chunk, 0)
    plsc.subcore_barrier()

    @pl.when(s == 0)
    def _():
        def wb(k, carry):
            pltpu.sync_copy(den_sp.at[pl.ds(k * 4352, 4352)], stage_v)
            pltpu.sync_copy(stage_v,
                            den_hbm.at[pl.ds(c * NP + k * 4352, 4352)])
            return carry

        lax.fori_loop(0, NP // 4352, wb, 0)


@functools.partial(
    pl.kernel,
    out_type=jax.ShapeDtypeStruct((2 * NP, H), jnp.float32),
    mesh=_mesh(),
    compiler_params=pltpu.CompilerParams(needs_layout_passes=False, use_tc_tiling_on_sc=False),
    scratch_types=[
        pltpu.VMEM((B_CH, 128), jnp.int32),
        pltpu.VMEM((B_CH, 128), jnp.int32),
        pltpu.VMEM((B_CH, 128), jnp.float32),
        pltpu.VMEM((B_CH * 128, H), jnp.float32),
        pltpu.VMEM((B_STG, H), jnp.float32),
        pltpu.VMEM_SHARED((NP, H), jnp.float32),
        pltpu.SemaphoreType.DMA,
    ],
)
def _sc_b(hA_hbm, hB_hbm, src_hbm, dst_hbm, w_hbm, num_hbm,
          src_v, dst_v, w_v, rows_v, stage_v, num_sp, sem):
    c = lax.axis_index("c")
    s = lax.axis_index("s")
    z16 = jnp.zeros((16,), jnp.float32)

    def zero(r, carry):
        stage_v[r, :] = z16
        return carry

    lax.fori_loop(0, B_STG, zero, 0)

    def zcp(k, carry):
        pltpu.sync_copy(stage_v, num_sp.at[pl.ds(s * NPS + k * B_STG, B_STG)])
        return carry

    lax.fori_loop(0, NPS // B_STG, zcp, 0)
    plsc.subcore_barrier()
    li = lax.iota(jnp.int32, 16)

    def chunk(t, carry):
        rowbase = s * B_ROWS + t * B_CH
        cps_in = [
            pltpu.async_copy(src_hbm.at[pl.ds(rowbase, B_CH)], src_v, sem),
            pltpu.async_copy(dst_hbm.at[pl.ds(rowbase, B_CH)], dst_v, sem),
            pltpu.async_copy(w_hbm.at[pl.ds(rowbase, B_CH)], w_v, sem),
        ]
        for cp in cps_in:
            cp.wait()

        @pl.when(c == 0)
        def _():
            cps = [pltpu.async_copy(hA_hbm.at[src_v.at[j]],
                                    rows_v.at[pl.ds(j * 128, 128)], sem)
                   for j in range(B_CH)]
            for cp in cps:
                cp.wait()

        @pl.when(c == 1)
        def _():
            cps = [pltpu.async_copy(hB_hbm.at[src_v.at[j]],
                                    rows_v.at[pl.ds(j * 128, 128)], sem)
                   for j in range(B_CH)]
            for cp in cps:
                cp.wait()

        def scale(j, carry2):
            for l in range(8):
                w16 = w_v[j, pl.ds(l * 16, 16)]
                ri = (j * 128 + l * 16) + li
                for f in range(H):
                    cf = jnp.full((16,), f, jnp.int32)
                    g = plsc.load_gather(rows_v, [ri, cf])
                    plsc.store_scatter(rows_v, [ri, cf], g * w16)
            return carry2

        lax.fori_loop(0, B_CH, scale, 0)

        cps_out = [pltpu.async_copy(rows_v.at[pl.ds(j * 128, 128)],
                                    num_sp.at[dst_v.at[j]], sem, add=True)
                   for j in range(B_CH)]
        for cp in cps_out:
            cp.wait()
        return carry

    lax.fori_loop(0, B_NCH, chunk, 0)
    plsc.subcore_barrier()
    def wb(k, carry):
        off = s * NPS + k * B_STG
        pltpu.sync_copy(num_sp.at[pl.ds(off, B_STG)], stage_v)
        pltpu.sync_copy(stage_v, num_hbm.at[pl.ds(c * NP + off, B_STG)])
        return carry

    lax.fori_loop(0, NPS // B_STG, wb, 0)


def _tc2_body(hA_ref, hB_ref, as_ref, ad_ref, d0_ref, d1_ref, n0_ref, n1_ref,
              ms_ref, md_ref, b_ref, o_ref):
    a_s = as_ref[...]
    a_d = ad_ref[...]
    M = jnp.maximum(ms_ref[0, 0] + md_ref[0, 0], 0.0)
    z = a_s + a_d
    lr = jnp.where(z >= 0, z, 0.2 * z)
    wself = jnp.exp(lr - M)
    den = d0_ref[...] + d1_ref[...] + wself + 1e-16
    nA = (n0_ref[...] + wself * hA_ref[...]) / den
    nB = (n1_ref[...] + wself * hB_ref[...]) / den
    o_ref[...] = jnp.concatenate([nA, nB], axis=1) + b_ref[0:1, :]


def _tc2(hA, hB, as_col, ad_col, d0, d1, n0, n1, ms, md, bias8):
    col = pl.BlockSpec((BN, 1), lambda i: (i, 0))
    half = pl.BlockSpec((BN, H), lambda i: (i, 0))
    return pl.pallas_call(
        _tc2_body,
        grid=(G1,),
        in_specs=[
            half, half, col, col, col, col, half, half,
            pl.BlockSpec((1, 1), lambda i: (0, 0), memory_space=pltpu.SMEM),
            pl.BlockSpec((1, 1), lambda i: (0, 0), memory_space=pltpu.SMEM),
            pl.BlockSpec((8, D), lambda i: (0, 0)),
        ],
        out_specs=pl.BlockSpec((BN, D), lambda i: (i, 0)),
        out_shape=jax.ShapeDtypeStruct((NP, D), jnp.float32),
    )(hA, hB, as_col, ad_col, d0, d1, n0, n1, ms, md, bias8)


def kernel(x, edge_index, W, att_src, att_dst, bias):
    src = edge_index[0].astype(jnp.int32)
    dst = edge_index[1].astype(jnp.int32)

    xp = jnp.concatenate([x.astype(jnp.float32),
                          jnp.zeros((NP - N, D), jnp.float32)], axis=0)
    sel8 = jnp.concatenate([att_src[None], att_dst[None],
                            jnp.zeros((6, D), jnp.float32)], axis=0)
    bias8 = jnp.broadcast_to(bias[None, :], (8, D)).astype(jnp.float32)

    hA, hB, asH, adH, ms, md = _tc1(xp, W.astype(jnp.float32), sel8)

    pad = EP - E
    srcP = jnp.concatenate([src, jnp.zeros((pad,), jnp.int32)]).reshape(EPR, 128)
    dstP = jnp.concatenate([dst, jnp.full((pad,), N, jnp.int32)]).reshape(EPR, 128)

    M16 = jnp.broadcast_to(jnp.maximum(ms[0, 0] + md[0, 0], 0.0), (16,))

    p_edge = _sc_a1(asH.reshape(NP), srcP)
    w_edge, den = _sc_a2(adH.reshape(NP), dstP, p_edge, M16)
    num = _sc_b(hA, hB, srcP, dstP, w_edge)

    as_col = asH.reshape(NP, 1)
    ad_col = adH.reshape(NP, 1)
    d0 = den[:NP].reshape(NP, 1)
    d1 = den[NP:].reshape(NP, 1)
    n0 = num[:NP]
    n1 = num[NP:]

    out = _tc2(hA, hB, as_col, ad_col, d0, d1, n0, n1, ms, md, bias8)
    return out[:N]


# speedup vs baseline: 41.5956x; 1.5349x over previous
import functools

import jax
import jax.numpy as jnp
from jax import lax
from jax.experimental import pallas as pl
from jax.experimental.pallas import tpu as pltpu
from jax.experimental.pallas import tpu_sc as plsc

N = 100000
D = 32
H = D // 2
E = 1600000
NP = 100096
NPS = NP // 16

EP = 1638400
EPR = EP // 128

A_ROWS = EPR // 32
A_CH = 16
A_NCH = A_ROWS // A_CH

B_ROWS = EPR // 16
B_CH = 8
B_NCH = B_ROWS // B_CH
B_STG = 368

BN = 4352
G1 = NP // BN

_mesh = lambda: plsc.VectorSubcoreMesh(
    core_axis_name="c", subcore_axis_name="s", num_cores=2, num_subcores=16)


def _tc1_body(x_ref, w_ref, sel_ref, hA_ref, hB_ref, as_ref, ad_ref,
              ms_ref, md_ref):
    i = pl.program_id(0)
    h = jnp.dot(x_ref[...], w_ref[...], preferred_element_type=jnp.float32)
    hA_ref[...] = h[:, :H]
    hB_ref[...] = h[:, H:]
    a8 = lax.dot_general(sel_ref[...], h, (((1,), (1,)), ((), ())),
                         preferred_element_type=jnp.float32)
    as_ref[...] = a8[0:1, :]
    ad_ref[...] = a8[1:2, :]
    bs = jnp.max(a8[0:1, :])
    bd = jnp.max(a8[1:2, :])

    @pl.when(i == 0)
    def _():
        ms_ref[0, 0] = bs
        md_ref[0, 0] = bd

    @pl.when(i > 0)
    def _():
        ms_ref[0, 0] = jnp.maximum(ms_ref[0, 0], bs)
        md_ref[0, 0] = jnp.maximum(md_ref[0, 0], bd)


def _tc1(xp, W, sel8):
    return pl.pallas_call(
        _tc1_body,
        grid=(G1,),
        in_specs=[
            pl.BlockSpec((BN, D), lambda i: (i, 0)),
            pl.BlockSpec((D, D), lambda i: (0, 0)),
            pl.BlockSpec((8, D), lambda i: (0, 0)),
        ],
        out_specs=[
            pl.BlockSpec((BN, H), lambda i: (i, 0)),
            pl.BlockSpec((BN, H), lambda i: (i, 0)),
            pl.BlockSpec((1, BN), lambda i: (0, i)),
            pl.BlockSpec((1, BN), lambda i: (0, i)),
            pl.BlockSpec((1, 1), lambda i: (0, 0), memory_space=pltpu.SMEM),
            pl.BlockSpec((1, 1), lambda i: (0, 0), memory_space=pltpu.SMEM),
        ],
        out_shape=[
            jax.ShapeDtypeStruct((NP, H), jnp.float32),
            jax.ShapeDtypeStruct((NP, H), jnp.float32),
            jax.ShapeDtypeStruct((1, NP), jnp.float32),
            jax.ShapeDtypeStruct((1, NP), jnp.float32),
            jax.ShapeDtypeStruct((1, 1), jnp.float32),
            jax.ShapeDtypeStruct((1, 1), jnp.float32),
        ],
    )(xp, W, sel8)


@functools.partial(
    pl.kernel,
    out_type=jax.ShapeDtypeStruct((EPR, 128), jnp.float32),
    mesh=_mesh(),
    compiler_params=pltpu.CompilerParams(needs_layout_passes=False, use_tc_tiling_on_sc=False),
    scratch_types=[
        pltpu.VMEM((NP,), jnp.float32),
        pltpu.VMEM((A_CH, 128), jnp.int32),
        pltpu.VMEM((A_CH, 128), jnp.float32),
    ],
)
def _sc_a1(asrc_hbm, src_hbm, p_hbm, tab_v, idx_v, p_v):
    c = lax.axis_index("c")
    s = lax.axis_index("s")
    wid = s * 2 + c
    pltpu.sync_copy(asrc_hbm, tab_v)

    def chunk(t, carry):
        rowbase = wid * A_ROWS + t * A_CH
        pltpu.sync_copy(src_hbm.at[pl.ds(rowbase, A_CH)], idx_v)

        def vec(j, carry2):
            for l in range(8):
                i16 = idx_v[j, pl.ds(l * 16, 16)]
                p_v[j, pl.ds(l * 16, 16)] = plsc.load_gather(tab_v, [i16])
            return carry2

        lax.fori_loop(0, A_CH, vec, 0)
        pltpu.sync_copy(p_v, p_hbm.at[pl.ds(rowbase, A_CH)])
        return carry

    lax.fori_loop(0, A_NCH, chunk, 0)


@functools.partial(
    pl.kernel,
    out_type=[
        jax.ShapeDtypeStruct((EPR, 128), jnp.float32),
        jax.ShapeDtypeStruct((2 * NP,), jnp.float32),
    ],
    mesh=_mesh(),
    compiler_params=pltpu.CompilerParams(needs_layout_passes=False, use_tc_tiling_on_sc=False),
    scratch_types=[
        pltpu.VMEM((NP,), jnp.float32),
        pltpu.VMEM((A_CH, 128), jnp.int32),
        pltpu.VMEM((A_CH, 128), jnp.float32),
        pltpu.VMEM((A_CH, 128), jnp.float32),
        pltpu.VMEM((16,), jnp.float32),
        pltpu.VMEM((4352,), jnp.float32),
        pltpu.VMEM_SHARED((NP,), jnp.float32),
        pltpu.SemaphoreType.DMA,
    ],
)
def _sc_a2(adst_hbm, dst_hbm, p_hbm, m_hbm, w_hbm, den_hbm,
           tab_v, idx_v, p_v, w_v, m_v, stage_v, den_sp, sem_a):
    c = lax.axis_index("c")
    s = lax.axis_index("s")
    wid = s * 2 + c
    z16 = jnp.zeros((16,), jnp.float32)

    @pl.when(s == 0)
    def _():
        def zero(r, carry):
            stage_v[pl.ds(r * 16, 16)] = z16
            return carry

        lax.fori_loop(0, 4352 // 16, zero, 0)

        def zcp(k, carry):
            pltpu.sync_copy(stage_v, den_sp.at[pl.ds(k * 4352, 4352)])
            return carry

        lax.fori_loop(0, NP // 4352, zcp, 0)

    pltpu.sync_copy(adst_hbm, tab_v)
    pltpu.sync_copy(m_hbm, m_v)
    plsc.subcore_barrier()
    mv = m_v[...]

    def chunk(t, carry):
        rowbase = wid * A_ROWS + t * A_CH
        pltpu.sync_copy(dst_hbm.at[pl.ds(rowbase, A_CH)], idx_v)
        pltpu.sync_copy(p_hbm.at[pl.ds(rowbase, A_CH)], p_v)

        def vec(j, carry2):
            for l in range(8):
                i16 = idx_v[j, pl.ds(l * 16, 16)]
                q = plsc.load_gather(tab_v, [i16])
                z = p_v[j, pl.ds(l * 16, 16)] + q
                lr = jnp.where(z >= 0, z, 0.2 * z)
                w_v[j, pl.ds(l * 16, 16)] = jnp.exp(lr - mv)
            return carry2

        lax.fori_loop(0, A_CH, vec, 0)
        pltpu.sync_copy(w_v, w_hbm.at[pl.ds(rowbase, A_CH)])

        cps = [pltpu.async_copy(w_v.at[j], den_sp.at[idx_v.at[j]],
                                sem_a, add=True)
               for j in range(A_CH)]
        for cp in cps:
            cp.wait()
        return carry

    lax.fori_loop(0, A_NCH, chunk, 0)
    plsc.subcore_barrier()

    @pl.when(s == 0)
    def _():
        def wb(k, carry):
            pltpu.sync_copy(den_sp.at[pl.ds(k * 4352, 4352)], stage_v)
            pltpu.sync_copy(stage_v,
                            den_hbm.at[pl.ds(c * NP + k * 4352, 4352)])
            return carry

        lax.fori_loop(0, NP // 4352, wb, 0)


@functools.partial(
    pl.kernel,
    out_type=jax.ShapeDtypeStruct((2 * NP, H), jnp.float32),
    mesh=_mesh(),
    compiler_params=pltpu.CompilerParams(needs_layout_passes=False, use_tc_tiling_on_sc=False),
    scratch_types=[
        pltpu.VMEM((B_CH, 128), jnp.int32),
        pltpu.VMEM((B_CH, 128), jnp.int32),
        pltpu.VMEM((B_CH, 128), jnp.float32),
        pltpu.VMEM((B_CH * 128, H), jnp.float32),
        pltpu.VMEM((B_STG, H), jnp.float32),
        pltpu.VMEM_SHARED((NP, H), jnp.float32),
        pltpu.SemaphoreType.DMA,
    ],
)
def _sc_b(hA_hbm, hB_hbm, src_hbm, dst_hbm, w_hbm, num_hbm,
          src_v, dst_v, w_v, rows_v, stage_v, num_sp, sem):
    c = lax.axis_index("c")
    s = lax.axis_index("s")
    z16 = jnp.zeros((16,), jnp.float32)

    def zero(r, carry):
        stage_v[r, :] = z16
        return carry

    lax.fori_loop(0, B_STG, zero, 0)

    def zcp(k, carry):
        pltpu.sync_copy(stage_v, num_sp.at[pl.ds(s * NPS + k * B_STG, B_STG)])
        return carry

    lax.fori_loop(0, NPS // B_STG, zcp, 0)
    plsc.subcore_barrier()
    li = lax.iota(jnp.int32, 16)

    def chunk(t, carry):
        rowbase = s * B_ROWS + t * B_CH
        cps_in = [
            pltpu.async_copy(src_hbm.at[pl.ds(rowbase, B_CH)], src_v, sem),
            pltpu.async_copy(dst_hbm.at[pl.ds(rowbase, B_CH)], dst_v, sem),
            pltpu.async_copy(w_hbm.at[pl.ds(rowbase, B_CH)], w_v, sem),
        ]
        for cp in cps_in:
            cp.wait()

        @pl.when(c == 0)
        def _():
            cps = [pltpu.async_copy(hA_hbm.at[src_v.at[j]],
                                    rows_v.at[pl.ds(j * 128, 128)], sem)
                   for j in range(B_CH)]
            for cp in cps:
                cp.wait()

        @pl.when(c == 1)
        def _():
            cps = [pltpu.async_copy(hB_hbm.at[src_v.at[j]],
                                    rows_v.at[pl.ds(j * 128, 128)], sem)
                   for j in range(B_CH)]
            for cp in cps:
                cp.wait()

        def scale(j, carry2):
            base = j * 128
            for l in range(8):
                w16 = w_v[j, pl.ds(l * 16, 16)]
                for e in range(16):
                    sp = w16[jnp.full((16,), e, jnp.int32)]
                    r = base + l * 16 + e
                    rows_v[r, :] = rows_v[r, :] * sp
            return carry2

        lax.fori_loop(0, B_CH, scale, 0)

        cps_out = [pltpu.async_copy(rows_v.at[pl.ds(j * 128, 128)],
                                    num_sp.at[dst_v.at[j]], sem, add=True)
                   for j in range(B_CH)]
        for cp in cps_out:
            cp.wait()
        return carry

    lax.fori_loop(0, B_NCH, chunk, 0)
    plsc.subcore_barrier()
    def wb(k, carry):
        off = s * NPS + k * B_STG
        pltpu.sync_copy(num_sp.at[pl.ds(off, B_STG)], stage_v)
        pltpu.sync_copy(stage_v, num_hbm.at[pl.ds(c * NP + off, B_STG)])
        return carry

    lax.fori_loop(0, NPS // B_STG, wb, 0)


def _tc2_body(hA_ref, hB_ref, as_ref, ad_ref, d0_ref, d1_ref, n0_ref, n1_ref,
              ms_ref, md_ref, b_ref, o_ref):
    a_s = as_ref[...]
    a_d = ad_ref[...]
    M = jnp.maximum(ms_ref[0, 0] + md_ref[0, 0], 0.0)
    z = a_s + a_d
    lr = jnp.where(z >= 0, z, 0.2 * z)
    wself = jnp.exp(lr - M)
    den = d0_ref[...] + d1_ref[...] + wself + 1e-16
    nA = (n0_ref[...] + wself * hA_ref[...]) / den
    nB = (n1_ref[...] + wself * hB_ref[...]) / den
    o_ref[...] = jnp.concatenate([nA, nB], axis=1) + b_ref[0:1, :]


def _tc2(hA, hB, as_col, ad_col, d0, d1, n0, n1, ms, md, bias8):
    col = pl.BlockSpec((BN, 1), lambda i: (i, 0))
    half = pl.BlockSpec((BN, H), lambda i: (i, 0))
    return pl.pallas_call(
        _tc2_body,
        grid=(G1,),
        in_specs=[
            half, half, col, col, col, col, half, half,
            pl.BlockSpec((1, 1), lambda i: (0, 0), memory_space=pltpu.SMEM),
            pl.BlockSpec((1, 1), lambda i: (0, 0), memory_space=pltpu.SMEM),
            pl.BlockSpec((8, D), lambda i: (0, 0)),
        ],
        out_specs=pl.BlockSpec((BN, D), lambda i: (i, 0)),
        out_shape=jax.ShapeDtypeStruct((NP, D), jnp.float32),
    )(hA, hB, as_col, ad_col, d0, d1, n0, n1, ms, md, bias8)


def kernel(x, edge_index, W, att_src, att_dst, bias):
    src = edge_index[0].astype(jnp.int32)
    dst = edge_index[1].astype(jnp.int32)

    xp = jnp.concatenate([x.astype(jnp.float32),
                          jnp.zeros((NP - N, D), jnp.float32)], axis=0)
    sel8 = jnp.concatenate([att_src[None], att_dst[None],
                            jnp.zeros((6, D), jnp.float32)], axis=0)
    bias8 = jnp.broadcast_to(bias[None, :], (8, D)).astype(jnp.float32)

    hA, hB, asH, adH, ms, md = _tc1(xp, W.astype(jnp.float32), sel8)

    pad = EP - E
    srcP = jnp.concatenate([src, jnp.zeros((pad,), jnp.int32)]).reshape(EPR, 128)
    dstP = jnp.concatenate([dst, jnp.full((pad,), N, jnp.int32)]).reshape(EPR, 128)

    M16 = jnp.broadcast_to(jnp.maximum(ms[0, 0] + md[0, 0], 0.0), (16,))

    p_edge = _sc_a1(asH.reshape(NP), srcP)
    w_edge, den = _sc_a2(adH.reshape(NP), dstP, p_edge, M16)
    num = _sc_b(hA, hB, srcP, dstP, w_edge)

    as_col = asH.reshape(NP, 1)
    ad_col = adH.reshape(NP, 1)
    d0 = den[:NP].reshape(NP, 1)
    d1 = den[NP:].reshape(NP, 1)
    n0 = num[:NP]
    n1 = num[NP:]

    out = _tc2(hA, hB, as_col, ad_col, d0, d1, n0, n1, ms, md, bias8)
    return out[:N]


# speedup vs baseline: 42.1879x; 1.0142x over previous
import functools

import jax
import jax.numpy as jnp
from jax import lax
from jax.experimental import pallas as pl
from jax.experimental.pallas import tpu as pltpu
from jax.experimental.pallas import tpu_sc as plsc

N = 100000
D = 32
H = D // 2
E = 1600000
NP = 100096
NPS = NP // 16

EP = 1638400
EPR = EP // 128

A_ROWS = EPR // 32
A_CH = 16
A_NCH = A_ROWS // A_CH

B_ROWS = EPR // 16
B_CH = 10
B_NCH = B_ROWS // B_CH
B_STG = 368

BN = 4352
G1 = NP // BN

_mesh = lambda: plsc.VectorSubcoreMesh(
    core_axis_name="c", subcore_axis_name="s", num_cores=2, num_subcores=16)


def _tc1_body(x_ref, w_ref, sel_ref, hA_ref, hB_ref, as_ref, ad_ref,
              ms_ref, md_ref):
    i = pl.program_id(0)
    h = jnp.dot(x_ref[...], w_ref[...], preferred_element_type=jnp.float32)
    hA_ref[...] = h[:, :H]
    hB_ref[...] = h[:, H:]
    a8 = lax.dot_general(sel_ref[...], h, (((1,), (1,)), ((), ())),
                         preferred_element_type=jnp.float32)
    as_ref[...] = a8[0:1, :]
    ad_ref[...] = a8[1:2, :]
    bs = jnp.max(a8[0:1, :])
    bd = jnp.max(a8[1:2, :])

    @pl.when(i == 0)
    def _():
        ms_ref[0, 0] = bs
        md_ref[0, 0] = bd

    @pl.when(i > 0)
    def _():
        ms_ref[0, 0] = jnp.maximum(ms_ref[0, 0], bs)
        md_ref[0, 0] = jnp.maximum(md_ref[0, 0], bd)


def _tc1(xp, W, sel8):
    return pl.pallas_call(
        _tc1_body,
        grid=(G1,),
        in_specs=[
            pl.BlockSpec((BN, D), lambda i: (i, 0)),
            pl.BlockSpec((D, D), lambda i: (0, 0)),
            pl.BlockSpec((8, D), lambda i: (0, 0)),
        ],
        out_specs=[
            pl.BlockSpec((BN, H), lambda i: (i, 0)),
            pl.BlockSpec((BN, H), lambda i: (i, 0)),
            pl.BlockSpec((1, BN), lambda i: (0, i)),
            pl.BlockSpec((1, BN), lambda i: (0, i)),
            pl.BlockSpec((1, 1), lambda i: (0, 0), memory_space=pltpu.SMEM),
            pl.BlockSpec((1, 1), lambda i: (0, 0), memory_space=pltpu.SMEM),
        ],
        out_shape=[
            jax.ShapeDtypeStruct((NP, H), jnp.float32),
            jax.ShapeDtypeStruct((NP, H), jnp.float32),
            jax.ShapeDtypeStruct((1, NP), jnp.float32),
            jax.ShapeDtypeStruct((1, NP), jnp.float32),
            jax.ShapeDtypeStruct((1, 1), jnp.float32),
            jax.ShapeDtypeStruct((1, 1), jnp.float32),
        ],
    )(xp, W, sel8)


@functools.partial(
    pl.kernel,
    out_type=jax.ShapeDtypeStruct((EPR, 128), jnp.float32),
    mesh=_mesh(),
    compiler_params=pltpu.CompilerParams(needs_layout_passes=False, use_tc_tiling_on_sc=False),
    scratch_types=[
        pltpu.VMEM((NP,), jnp.float32),
        pltpu.VMEM((A_CH, 128), jnp.int32),
        pltpu.VMEM((A_CH, 128), jnp.float32),
    ],
)
def _sc_a1(asrc_hbm, src_hbm, p_hbm, tab_v, idx_v, p_v):
    c = lax.axis_index("c")
    s = lax.axis_index("s")
    wid = s * 2 + c
    pltpu.sync_copy(asrc_hbm, tab_v)

    def chunk(t, carry):
        rowbase = wid * A_ROWS + t * A_CH
        pltpu.sync_copy(src_hbm.at[pl.ds(rowbase, A_CH)], idx_v)

        def vec(j, carry2):
            for l in range(8):
                i16 = idx_v[j, pl.ds(l * 16, 16)]
                p_v[j, pl.ds(l * 16, 16)] = plsc.load_gather(tab_v, [i16])
            return carry2

        lax.fori_loop(0, A_CH, vec, 0)
        pltpu.sync_copy(p_v, p_hbm.at[pl.ds(rowbase, A_CH)])
        return carry

    lax.fori_loop(0, A_NCH, chunk, 0)


@functools.partial(
    pl.kernel,
    out_type=[
        jax.ShapeDtypeStruct((EPR, 128), jnp.float32),
        jax.ShapeDtypeStruct((2 * NP,), jnp.float32),
    ],
    mesh=_mesh(),
    compiler_params=pltpu.CompilerParams(needs_layout_passes=False, use_tc_tiling_on_sc=False),
    scratch_types=[
        pltpu.VMEM((NP,), jnp.float32),
        pltpu.VMEM((A_CH, 128), jnp.int32),
        pltpu.VMEM((A_CH, 128), jnp.float32),
        pltpu.VMEM((A_CH, 128), jnp.float32),
        pltpu.VMEM((16,), jnp.float32),
        pltpu.VMEM((4352,), jnp.float32),
        pltpu.VMEM_SHARED((NP,), jnp.float32),
        pltpu.SemaphoreType.DMA,
    ],
)
def _sc_a2(adst_hbm, dst_hbm, p_hbm, m_hbm, w_hbm, den_hbm,
           tab_v, idx_v, p_v, w_v, m_v, stage_v, den_sp, sem_a):
    c = lax.axis_index("c")
    s = lax.axis_index("s")
    wid = s * 2 + c
    z16 = jnp.zeros((16,), jnp.float32)

    @pl.when(s == 0)
    def _():
        def zero(r, carry):
            stage_v[pl.ds(r * 16, 16)] = z16
            return carry

        lax.fori_loop(0, 4352 // 16, zero, 0)

        def zcp(k, carry):
            pltpu.sync_copy(stage_v, den_sp.at[pl.ds(k * 4352, 4352)])
            return carry

        lax.fori_loop(0, NP // 4352, zcp, 0)

    pltpu.sync_copy(adst_hbm, tab_v)
    pltpu.sync_copy(m_hbm, m_v)
    plsc.subcore_barrier()
    mv = m_v[...]

    def chunk(t, carry):
        rowbase = wid * A_ROWS + t * A_CH
        pltpu.sync_copy(dst_hbm.at[pl.ds(rowbase, A_CH)], idx_v)
        pltpu.sync_copy(p_hbm.at[pl.ds(rowbase, A_CH)], p_v)

        def vec(j, carry2):
            for l in range(8):
                i16 = idx_v[j, pl.ds(l * 16, 16)]
                q = plsc.load_gather(tab_v, [i16])
                z = p_v[j, pl.ds(l * 16, 16)] + q
                lr = jnp.where(z >= 0, z, 0.2 * z)
                w_v[j, pl.ds(l * 16, 16)] = jnp.exp(lr - mv)
            return carry2

        lax.fori_loop(0, A_CH, vec, 0)
        pltpu.sync_copy(w_v, w_hbm.at[pl.ds(rowbase, A_CH)])

        cps = [pltpu.async_copy(w_v.at[j], den_sp.at[idx_v.at[j]],
                                sem_a, add=True)
               for j in range(A_CH)]
        for cp in cps:
            cp.wait()
        return carry

    lax.fori_loop(0, A_NCH, chunk, 0)
    plsc.subcore_barrier()

    @pl.when(s == 0)
    def _():
        def wb(k, carry):
            pltpu.sync_copy(den_sp.at[pl.ds(k * 4352, 4352)], stage_v)
            pltpu.sync_copy(stage_v,
                            den_hbm.at[pl.ds(c * NP + k * 4352, 4352)])
            return carry

        lax.fori_loop(0, NP // 4352, wb, 0)


@functools.partial(
    pl.kernel,
    out_type=jax.ShapeDtypeStruct((2 * NP, H), jnp.float32),
    mesh=_mesh(),
    compiler_params=pltpu.CompilerParams(needs_layout_passes=False, use_tc_tiling_on_sc=False),
    scratch_types=[
        pltpu.VMEM((B_CH, 128), jnp.int32),
        pltpu.VMEM((B_CH, 128), jnp.int32),
        pltpu.VMEM((B_CH, 128), jnp.float32),
        pltpu.VMEM((B_CH * 128, H), jnp.float32),
        pltpu.VMEM_SHARED((NP, H), jnp.float32),
        pltpu.SemaphoreType.DMA,
    ],
)
def _sc_b(hA_hbm, hB_hbm, src_hbm, dst_hbm, w_hbm, num_hbm,
          src_v, dst_v, w_v, rows_v, num_sp, sem):
    c = lax.axis_index("c")
    s = lax.axis_index("s")
    z16 = jnp.zeros((16,), jnp.float32)

    def zero(r, carry):
        rows_v[r, :] = z16
        return carry

    lax.fori_loop(0, B_STG, zero, 0)

    def zcp(k, carry):
        pltpu.sync_copy(rows_v.at[pl.ds(0, B_STG)],
                        num_sp.at[pl.ds(s * NPS + k * B_STG, B_STG)])
        return carry

    lax.fori_loop(0, NPS // B_STG, zcp, 0)
    plsc.subcore_barrier()
    li = lax.iota(jnp.int32, 16)

    def chunk(t, carry):
        rowbase = s * B_ROWS + t * B_CH
        cps_in = [
            pltpu.async_copy(src_hbm.at[pl.ds(rowbase, B_CH)], src_v, sem),
            pltpu.async_copy(dst_hbm.at[pl.ds(rowbase, B_CH)], dst_v, sem),
            pltpu.async_copy(w_hbm.at[pl.ds(rowbase, B_CH)], w_v, sem),
        ]
        for cp in cps_in:
            cp.wait()

        @pl.when(c == 0)
        def _():
            cps = [pltpu.async_copy(hA_hbm.at[src_v.at[j]],
                                    rows_v.at[pl.ds(j * 128, 128)], sem)
                   for j in range(B_CH)]
            for cp in cps:
                cp.wait()

        @pl.when(c == 1)
        def _():
            cps = [pltpu.async_copy(hB_hbm.at[src_v.at[j]],
                                    rows_v.at[pl.ds(j * 128, 128)], sem)
                   for j in range(B_CH)]
            for cp in cps:
                cp.wait()

        def scale(j, carry2):
            base = j * 128
            for l in range(8):
                w16 = w_v[j, pl.ds(l * 16, 16)]
                for e in range(16):
                    sp = w16[jnp.full((16,), e, jnp.int32)]
                    r = base + l * 16 + e
                    rows_v[r, :] = rows_v[r, :] * sp
            return carry2

        lax.fori_loop(0, B_CH, scale, 0)

        cps_out = [pltpu.async_copy(rows_v.at[pl.ds(j * 128, 128)],
                                    num_sp.at[dst_v.at[j]], sem, add=True)
                   for j in range(B_CH)]
        for cp in cps_out:
            cp.wait()
        return carry

    lax.fori_loop(0, B_NCH, chunk, 0)
    plsc.subcore_barrier()
    def wb(k, carry):
        off = s * NPS + k * B_STG
        pltpu.sync_copy(num_sp.at[pl.ds(off, B_STG)], rows_v.at[pl.ds(0, B_STG)])
        pltpu.sync_copy(rows_v.at[pl.ds(0, B_STG)],
                        num_hbm.at[pl.ds(c * NP + off, B_STG)])
        return carry

    lax.fori_loop(0, NPS // B_STG, wb, 0)


def _tc2_body(hA_ref, hB_ref, as_ref, ad_ref, d0_ref, d1_ref, n0_ref, n1_ref,
              ms_ref, md_ref, b_ref, o_ref):
    a_s = as_ref[...]
    a_d = ad_ref[...]
    M = jnp.maximum(ms_ref[0, 0] + md_ref[0, 0], 0.0)
    z = a_s + a_d
    lr = jnp.where(z >= 0, z, 0.2 * z)
    wself = jnp.exp(lr - M)
    den = d0_ref[...] + d1_ref[...] + wself + 1e-16
    nA = (n0_ref[...] + wself * hA_ref[...]) / den
    nB = (n1_ref[...] + wself * hB_ref[...]) / den
    o_ref[...] = jnp.concatenate([nA, nB], axis=1) + b_ref[0:1, :]


def _tc2(hA, hB, as_col, ad_col, d0, d1, n0, n1, ms, md, bias8):
    col = pl.BlockSpec((BN, 1), lambda i: (i, 0))
    half = pl.BlockSpec((BN, H), lambda i: (i, 0))
    return pl.pallas_call(
        _tc2_body,
        grid=(G1,),
        in_specs=[
            half, half, col, col, col, col, half, half,
            pl.BlockSpec((1, 1), lambda i: (0, 0), memory_space=pltpu.SMEM),
            pl.BlockSpec((1, 1), lambda i: (0, 0), memory_space=pltpu.SMEM),
            pl.BlockSpec((8, D), lambda i: (0, 0)),
        ],
        out_specs=pl.BlockSpec((BN, D), lambda i: (i, 0)),
        out_shape=jax.ShapeDtypeStruct((NP, D), jnp.float32),
    )(hA, hB, as_col, ad_col, d0, d1, n0, n1, ms, md, bias8)


def kernel(x, edge_index, W, att_src, att_dst, bias):
    src = edge_index[0].astype(jnp.int32)
    dst = edge_index[1].astype(jnp.int32)

    xp = jnp.concatenate([x.astype(jnp.float32),
                          jnp.zeros((NP - N, D), jnp.float32)], axis=0)
    sel8 = jnp.concatenate([att_src[None], att_dst[None],
                            jnp.zeros((6, D), jnp.float32)], axis=0)
    bias8 = jnp.broadcast_to(bias[None, :], (8, D)).astype(jnp.float32)

    hA, hB, asH, adH, ms, md = _tc1(xp, W.astype(jnp.float32), sel8)

    pad = EP - E
    srcP = jnp.concatenate([src, jnp.zeros((pad,), jnp.int32)]).reshape(EPR, 128)
    dstP = jnp.concatenate([dst, jnp.full((pad,), N, jnp.int32)]).reshape(EPR, 128)

    M16 = jnp.broadcast_to(jnp.maximum(ms[0, 0] + md[0, 0], 0.0), (16,))

    p_edge = _sc_a1(asH.reshape(NP), srcP)
    w_edge, den = _sc_a2(adH.reshape(NP), dstP, p_edge, M16)
    num = _sc_b(hA, hB, srcP, dstP, w_edge)

    as_col = asH.reshape(NP, 1)
    ad_col = adH.reshape(NP, 1)
    d0 = den[:NP].reshape(NP, 1)
    d1 = den[NP:].reshape(NP, 1)
    n0 = num[:NP]
    n1 = num[NP:]

    out = _tc2(hA, hB, as_col, ad_col, d0, d1, n0, n1, ms, md, bias8)
    return out[:N]


# speedup vs baseline: 46.9085x; 1.1119x over previous
import functools

import jax
import jax.numpy as jnp
from jax import lax
from jax.experimental import pallas as pl
from jax.experimental.pallas import tpu as pltpu
from jax.experimental.pallas import tpu_sc as plsc

N = 100000
D = 32
H = D // 2
E = 1600000
NP = 100096
NPS = NP // 16

EP = 1638400
EPR = EP // 128

A_ROWS = EPR // 32
A_CH = 16
A_NCH = A_ROWS // A_CH

B_ROWS = EPR // 16
B_CH = 4
B_NCH = B_ROWS // B_CH
B_STG = 368

BN = 4352
G1 = NP // BN

_mesh = lambda: plsc.VectorSubcoreMesh(
    core_axis_name="c", subcore_axis_name="s", num_cores=2, num_subcores=16)


def _tc1_body(x_ref, w_ref, sel_ref, hA_ref, hB_ref, as_ref, ad_ref,
              ms_ref, md_ref):
    i = pl.program_id(0)
    h = jnp.dot(x_ref[...], w_ref[...], preferred_element_type=jnp.float32)
    hA_ref[...] = h[:, :H]
    hB_ref[...] = h[:, H:]
    a8 = lax.dot_general(sel_ref[...], h, (((1,), (1,)), ((), ())),
                         preferred_element_type=jnp.float32)
    as_ref[...] = a8[0:1, :]
    ad_ref[...] = a8[1:2, :]
    bs = jnp.max(a8[0:1, :])
    bd = jnp.max(a8[1:2, :])

    @pl.when(i == 0)
    def _():
        ms_ref[0, 0] = bs
        md_ref[0, 0] = bd

    @pl.when(i > 0)
    def _():
        ms_ref[0, 0] = jnp.maximum(ms_ref[0, 0], bs)
        md_ref[0, 0] = jnp.maximum(md_ref[0, 0], bd)


def _tc1(xp, W, sel8):
    return pl.pallas_call(
        _tc1_body,
        grid=(G1,),
        in_specs=[
            pl.BlockSpec((BN, D), lambda i: (i, 0)),
            pl.BlockSpec((D, D), lambda i: (0, 0)),
            pl.BlockSpec((8, D), lambda i: (0, 0)),
        ],
        out_specs=[
            pl.BlockSpec((BN, H), lambda i: (i, 0)),
            pl.BlockSpec((BN, H), lambda i: (i, 0)),
            pl.BlockSpec((1, BN), lambda i: (0, i)),
            pl.BlockSpec((1, BN), lambda i: (0, i)),
            pl.BlockSpec((1, 1), lambda i: (0, 0), memory_space=pltpu.SMEM),
            pl.BlockSpec((1, 1), lambda i: (0, 0), memory_space=pltpu.SMEM),
        ],
        out_shape=[
            jax.ShapeDtypeStruct((NP, H), jnp.float32),
            jax.ShapeDtypeStruct((NP, H), jnp.float32),
            jax.ShapeDtypeStruct((1, NP), jnp.float32),
            jax.ShapeDtypeStruct((1, NP), jnp.float32),
            jax.ShapeDtypeStruct((1, 1), jnp.float32),
            jax.ShapeDtypeStruct((1, 1), jnp.float32),
        ],
    )(xp, W, sel8)


@functools.partial(
    pl.kernel,
    out_type=jax.ShapeDtypeStruct((EPR, 128), jnp.float32),
    mesh=_mesh(),
    compiler_params=pltpu.CompilerParams(needs_layout_passes=False, use_tc_tiling_on_sc=False),
    scratch_types=[
        pltpu.VMEM((NP,), jnp.float32),
        pltpu.VMEM((A_CH, 128), jnp.int32),
        pltpu.VMEM((A_CH, 128), jnp.float32),
    ],
)
def _sc_a1(asrc_hbm, src_hbm, p_hbm, tab_v, idx_v, p_v):
    c = lax.axis_index("c")
    s = lax.axis_index("s")
    wid = s * 2 + c
    pltpu.sync_copy(asrc_hbm, tab_v)

    def chunk(t, carry):
        rowbase = wid * A_ROWS + t * A_CH
        pltpu.sync_copy(src_hbm.at[pl.ds(rowbase, A_CH)], idx_v)

        def vec(j, carry2):
            for l in range(8):
                i16 = idx_v[j, pl.ds(l * 16, 16)]
                p_v[j, pl.ds(l * 16, 16)] = plsc.load_gather(tab_v, [i16])
            return carry2

        lax.fori_loop(0, A_CH, vec, 0)
        pltpu.sync_copy(p_v, p_hbm.at[pl.ds(rowbase, A_CH)])
        return carry

    lax.fori_loop(0, A_NCH, chunk, 0)


@functools.partial(
    pl.kernel,
    out_type=[
        jax.ShapeDtypeStruct((EPR, 128), jnp.float32),
        jax.ShapeDtypeStruct((2 * NP,), jnp.float32),
    ],
    mesh=_mesh(),
    compiler_params=pltpu.CompilerParams(needs_layout_passes=False, use_tc_tiling_on_sc=False),
    scratch_types=[
        pltpu.VMEM((NP,), jnp.float32),
        pltpu.VMEM((A_CH, 128), jnp.int32),
        pltpu.VMEM((A_CH, 128), jnp.float32),
        pltpu.VMEM((A_CH, 128), jnp.float32),
        pltpu.VMEM((16,), jnp.float32),
        pltpu.VMEM((4352,), jnp.float32),
        pltpu.VMEM_SHARED((NP,), jnp.float32),
        pltpu.SemaphoreType.DMA,
    ],
)
def _sc_a2(adst_hbm, dst_hbm, p_hbm, m_hbm, w_hbm, den_hbm,
           tab_v, idx_v, p_v, w_v, m_v, stage_v, den_sp, sem_a):
    c = lax.axis_index("c")
    s = lax.axis_index("s")
    wid = s * 2 + c
    z16 = jnp.zeros((16,), jnp.float32)

    @pl.when(s == 0)
    def _():
        def zero(r, carry):
            stage_v[pl.ds(r * 16, 16)] = z16
            return carry

        lax.fori_loop(0, 4352 // 16, zero, 0)

        def zcp(k, carry):
            pltpu.sync_copy(stage_v, den_sp.at[pl.ds(k * 4352, 4352)])
            return carry

        lax.fori_loop(0, NP // 4352, zcp, 0)

    pltpu.sync_copy(adst_hbm, tab_v)
    pltpu.sync_copy(m_hbm, m_v)
    plsc.subcore_barrier()
    mv = m_v[...]

    def chunk(t, carry):
        rowbase = wid * A_ROWS + t * A_CH
        pltpu.sync_copy(dst_hbm.at[pl.ds(rowbase, A_CH)], idx_v)
        pltpu.sync_copy(p_hbm.at[pl.ds(rowbase, A_CH)], p_v)

        def vec(j, carry2):
            for l in range(8):
                i16 = idx_v[j, pl.ds(l * 16, 16)]
                q = plsc.load_gather(tab_v, [i16])
                z = p_v[j, pl.ds(l * 16, 16)] + q
                lr = jnp.where(z >= 0, z, 0.2 * z)
                w_v[j, pl.ds(l * 16, 16)] = jnp.exp(lr - mv)
            return carry2

        lax.fori_loop(0, A_CH, vec, 0)
        pltpu.sync_copy(w_v, w_hbm.at[pl.ds(rowbase, A_CH)])

        cps = [pltpu.async_copy(w_v.at[j], den_sp.at[idx_v.at[j]],
                                sem_a, add=True)
               for j in range(A_CH)]
        for cp in cps:
            cp.wait()
        return carry

    lax.fori_loop(0, A_NCH, chunk, 0)
    plsc.subcore_barrier()

    @pl.when(s == 0)
    def _():
        def wb(k, carry):
            pltpu.sync_copy(den_sp.at[pl.ds(k * 4352, 4352)], stage_v)
            pltpu.sync_copy(stage_v,
                            den_hbm.at[pl.ds(c * NP + k * 4352, 4352)])
            return carry

        lax.fori_loop(0, NP // 4352, wb, 0)


@functools.partial(
    pl.kernel,
    out_type=jax.ShapeDtypeStruct((2 * NP, H), jnp.float32),
    mesh=_mesh(),
    compiler_params=pltpu.CompilerParams(needs_layout_passes=False, use_tc_tiling_on_sc=False),
    scratch_types=[
        pltpu.VMEM((B_CH, 128), jnp.int32),
        pltpu.VMEM((B_CH, 128), jnp.int32),
        pltpu.VMEM((B_CH, 128), jnp.float32),
        pltpu.VMEM((B_CH * 128, H), jnp.float32),
        pltpu.VMEM((B_CH, 128), jnp.int32),
        pltpu.VMEM((B_CH, 128), jnp.int32),
        pltpu.VMEM((B_CH, 128), jnp.float32),
        pltpu.VMEM((B_CH * 128, H), jnp.float32),
        pltpu.VMEM_SHARED((NP, H), jnp.float32),
        pltpu.SemaphoreType.DMA,
        pltpu.SemaphoreType.DMA,
        pltpu.SemaphoreType.DMA,
        pltpu.SemaphoreType.DMA,
        pltpu.SemaphoreType.DMA,
        pltpu.SemaphoreType.DMA,
    ],
)
def _sc_b(hA_hbm, hB_hbm, src_hbm, dst_hbm, w_hbm, num_hbm,
          s0, d0, w0, r0, s1, d1, w1, r1, num_sp,
          semi0, semi1, semg0, semg1, semc0, semc1):
    c = lax.axis_index("c")
    s = lax.axis_index("s")
    z16 = jnp.zeros((16,), jnp.float32)

    def zero(r, carry):
        r0[r, :] = z16
        return carry

    lax.fori_loop(0, B_STG, zero, 0)

    def zcp(k, carry):
        pltpu.sync_copy(r0.at[pl.ds(0, B_STG)],
                        num_sp.at[pl.ds(s * NPS + k * B_STG, B_STG)])
        return carry

    lax.fori_loop(0, NPS // B_STG, zcp, 0)
    plsc.subcore_barrier()

    def fire_in(t, sv, dv, wv, sem):
        rowbase = s * B_ROWS + t * B_CH
        pltpu.async_copy(src_hbm.at[pl.ds(rowbase, B_CH)], sv, sem)
        pltpu.async_copy(dst_hbm.at[pl.ds(rowbase, B_CH)], dv, sem)
        pltpu.async_copy(w_hbm.at[pl.ds(rowbase, B_CH)], wv, sem)

    def drain_in(sv, dv, wv, sem):
        pltpu.make_async_copy(src_hbm.at[pl.ds(0, B_CH)], sv, sem).wait()
        pltpu.make_async_copy(dst_hbm.at[pl.ds(0, B_CH)], dv, sem).wait()
        pltpu.make_async_copy(w_hbm.at[pl.ds(0, B_CH)], wv, sem).wait()

    def fire_g(sv, rv, sem):
        @pl.when(c == 0)
        def _():
            for j in range(B_CH):
                pltpu.async_copy(hA_hbm.at[sv.at[j]],
                                 rv.at[pl.ds(j * 128, 128)], sem)

        @pl.when(c == 1)
        def _():
            for j in range(B_CH):
                pltpu.async_copy(hB_hbm.at[sv.at[j]],
                                 rv.at[pl.ds(j * 128, 128)], sem)

    def drain_g(rv, sem):
        for j in range(B_CH):
            pltpu.make_async_copy(hA_hbm.at[pl.ds(0, 128)],
                                  rv.at[pl.ds(j * 128, 128)], sem).wait()

    def fire_sc(rv, dv, sem):
        for j in range(B_CH):
            pltpu.async_copy(rv.at[pl.ds(j * 128, 128)],
                             num_sp.at[dv.at[j]], sem, add=True)

    def drain_sc(rv, sem):
        for j in range(B_CH):
            pltpu.make_async_copy(rv.at[pl.ds(j * 128, 128)],
                                  num_sp.at[pl.ds(0, 128)], sem).wait()

    def scale(wv, rv):
        def body(j, carry2):
            base = j * 128
            for l in range(8):
                w16 = wv[j, pl.ds(l * 16, 16)]
                for e in range(16):
                    sp = w16[jnp.full((16,), e, jnp.int32)]
                    r = base + l * 16 + e
                    rv[r, :] = rv[r, :] * sp
            return carry2

        lax.fori_loop(0, B_CH, body, 0)

    G2 = B_NCH // 2
    fire_in(0, s0, d0, w0, semi0)
    drain_in(s0, d0, w0, semi0)
    fire_g(s0, r0, semg0)

    def dchunk(g, carry):
        t0 = 2 * g

        @pl.when(g > 0)
        def _():
            drain_sc(r1, semc1)

        fire_in(t0 + 1, s1, d1, w1, semi1)
        drain_g(r0, semg0)
        scale(w0, r0)
        fire_sc(r0, d0, semc0)
        drain_in(s1, d1, w1, semi1)
        fire_g(s1, r1, semg1)
        drain_sc(r0, semc0)

        @pl.when(g + 1 < G2)
        def _():
            fire_in(t0 + 2, s0, d0, w0, semi0)
            drain_in(s0, d0, w0, semi0)
            fire_g(s0, r0, semg0)

        drain_g(r1, semg1)
        scale(w1, r1)
        fire_sc(r1, d1, semc1)
        return carry

    lax.fori_loop(0, G2, dchunk, 0)
    drain_sc(r1, semc1)
    plsc.subcore_barrier()
    def wb(k, carry):
        off = s * NPS + k * B_STG
        pltpu.sync_copy(num_sp.at[pl.ds(off, B_STG)], r0.at[pl.ds(0, B_STG)])
        pltpu.sync_copy(r0.at[pl.ds(0, B_STG)],
                        num_hbm.at[pl.ds(c * NP + off, B_STG)])
        return carry

    lax.fori_loop(0, NPS // B_STG, wb, 0)


def _tc2_body(hA_ref, hB_ref, as_ref, ad_ref, d0_ref, d1_ref, n0_ref, n1_ref,
              ms_ref, md_ref, b_ref, o_ref):
    a_s = as_ref[...]
    a_d = ad_ref[...]
    M = jnp.maximum(ms_ref[0, 0] + md_ref[0, 0], 0.0)
    z = a_s + a_d
    lr = jnp.where(z >= 0, z, 0.2 * z)
    wself = jnp.exp(lr - M)
    den = d0_ref[...] + d1_ref[...] + wself + 1e-16
    nA = (n0_ref[...] + wself * hA_ref[...]) / den
    nB = (n1_ref[...] + wself * hB_ref[...]) / den
    o_ref[...] = jnp.concatenate([nA, nB], axis=1) + b_ref[0:1, :]


def _tc2(hA, hB, as_col, ad_col, d0, d1, n0, n1, ms, md, bias8):
    col = pl.BlockSpec((BN, 1), lambda i: (i, 0))
    half = pl.BlockSpec((BN, H), lambda i: (i, 0))
    return pl.pallas_call(
        _tc2_body,
        grid=(G1,),
        in_specs=[
            half, half, col, col, col, col, half, half,
            pl.BlockSpec((1, 1), lambda i: (0, 0), memory_space=pltpu.SMEM),
            pl.BlockSpec((1, 1), lambda i: (0, 0), memory_space=pltpu.SMEM),
            pl.BlockSpec((8, D), lambda i: (0, 0)),
        ],
        out_specs=pl.BlockSpec((BN, D), lambda i: (i, 0)),
        out_shape=jax.ShapeDtypeStruct((NP, D), jnp.float32),
    )(hA, hB, as_col, ad_col, d0, d1, n0, n1, ms, md, bias8)


def kernel(x, edge_index, W, att_src, att_dst, bias):
    src = edge_index[0].astype(jnp.int32)
    dst = edge_index[1].astype(jnp.int32)

    xp = jnp.concatenate([x.astype(jnp.float32),
                          jnp.zeros((NP - N, D), jnp.float32)], axis=0)
    sel8 = jnp.concatenate([att_src[None], att_dst[None],
                            jnp.zeros((6, D), jnp.float32)], axis=0)
    bias8 = jnp.broadcast_to(bias[None, :], (8, D)).astype(jnp.float32)

    hA, hB, asH, adH, ms, md = _tc1(xp, W.astype(jnp.float32), sel8)

    pad = EP - E
    srcP = jnp.concatenate([src, jnp.zeros((pad,), jnp.int32)]).reshape(EPR, 128)
    dstP = jnp.concatenate([dst, jnp.full((pad,), N, jnp.int32)]).reshape(EPR, 128)

    M16 = jnp.broadcast_to(jnp.maximum(ms[0, 0] + md[0, 0], 0.0), (16,))

    p_edge = _sc_a1(asH.reshape(NP), srcP)
    w_edge, den = _sc_a2(adH.reshape(NP), dstP, p_edge, M16)
    num = _sc_b(hA, hB, srcP, dstP, w_edge)

    as_col = asH.reshape(NP, 1)
    ad_col = adH.reshape(NP, 1)
    d0 = den[:NP].reshape(NP, 1)
    d1 = den[NP:].reshape(NP, 1)
    n0 = num[:NP]
    n1 = num[NP:]

    out = _tc2(hA, hB, as_col, ad_col, d0, d1, n0, n1, ms, md, bias8)
    return out[:N]


# speedup vs baseline: 47.4459x; 1.0115x over previous
import functools

import jax
import jax.numpy as jnp
from jax import lax
from jax.experimental import pallas as pl
from jax.experimental.pallas import tpu as pltpu
from jax.experimental.pallas import tpu_sc as plsc

N = 100000
D = 32
H = D // 2
E = 1600000
NP = 100096
NPS = NP // 16

EP = 1638400
EPR = EP // 128

A_ROWS = EPR // 32
A_CH = 16
A_NCH = A_ROWS // A_CH

B_ROWS = EPR // 16
B_CH = 4
B_NCH = B_ROWS // B_CH
B_STG = 368

BN = 4352
G1 = NP // BN

_mesh = lambda: plsc.VectorSubcoreMesh(
    core_axis_name="c", subcore_axis_name="s", num_cores=2, num_subcores=16)


def _tc1_body(x_ref, w_ref, sel_ref, hA_ref, hB_ref, as_ref, ad_ref,
              ms_ref, md_ref):
    i = pl.program_id(0)
    h = jnp.dot(x_ref[...], w_ref[...], preferred_element_type=jnp.float32)
    hA_ref[...] = h[:, :H]
    hB_ref[...] = h[:, H:]
    a8 = lax.dot_general(sel_ref[...], h, (((1,), (1,)), ((), ())),
                         preferred_element_type=jnp.float32)
    as_ref[...] = a8[0:1, :]
    ad_ref[...] = a8[1:2, :]
    bs = jnp.max(a8[0:1, :])
    bd = jnp.max(a8[1:2, :])

    @pl.when(i == 0)
    def _():
        ms_ref[0, 0] = bs
        md_ref[0, 0] = bd

    @pl.when(i > 0)
    def _():
        ms_ref[0, 0] = jnp.maximum(ms_ref[0, 0], bs)
        md_ref[0, 0] = jnp.maximum(md_ref[0, 0], bd)


def _tc1(xp, W, sel8):
    return pl.pallas_call(
        _tc1_body,
        grid=(G1,),
        in_specs=[
            pl.BlockSpec((BN, D), lambda i: (i, 0)),
            pl.BlockSpec((D, D), lambda i: (0, 0)),
            pl.BlockSpec((8, D), lambda i: (0, 0)),
        ],
        out_specs=[
            pl.BlockSpec((BN, H), lambda i: (i, 0)),
            pl.BlockSpec((BN, H), lambda i: (i, 0)),
            pl.BlockSpec((1, BN), lambda i: (0, i)),
            pl.BlockSpec((1, BN), lambda i: (0, i)),
            pl.BlockSpec((1, 1), lambda i: (0, 0), memory_space=pltpu.SMEM),
            pl.BlockSpec((1, 1), lambda i: (0, 0), memory_space=pltpu.SMEM),
        ],
        out_shape=[
            jax.ShapeDtypeStruct((NP, H), jnp.float32),
            jax.ShapeDtypeStruct((NP, H), jnp.float32),
            jax.ShapeDtypeStruct((1, NP), jnp.float32),
            jax.ShapeDtypeStruct((1, NP), jnp.float32),
            jax.ShapeDtypeStruct((1, 1), jnp.float32),
            jax.ShapeDtypeStruct((1, 1), jnp.float32),
        ],
    )(xp, W, sel8)


@functools.partial(
    pl.kernel,
    out_type=jax.ShapeDtypeStruct((EPR, 128), jnp.float32),
    mesh=_mesh(),
    compiler_params=pltpu.CompilerParams(needs_layout_passes=False, use_tc_tiling_on_sc=False),
    scratch_types=[
        pltpu.VMEM((NP,), jnp.float32),
        pltpu.VMEM((A_CH, 128), jnp.int32),
        pltpu.VMEM((A_CH, 128), jnp.float32),
    ],
)
def _sc_a1(asrc_hbm, src_hbm, p_hbm, tab_v, idx_v, p_v):
    c = lax.axis_index("c")
    s = lax.axis_index("s")
    wid = s * 2 + c
    pltpu.sync_copy(asrc_hbm, tab_v)

    def chunk(t, carry):
        rowbase = wid * A_ROWS + t * A_CH
        pltpu.sync_copy(src_hbm.at[pl.ds(rowbase, A_CH)], idx_v)

        def vec(j, carry2):
            for l in range(8):
                i16 = idx_v[j, pl.ds(l * 16, 16)]
                p_v[j, pl.ds(l * 16, 16)] = plsc.load_gather(tab_v, [i16])
            return carry2

        lax.fori_loop(0, A_CH, vec, 0)
        pltpu.sync_copy(p_v, p_hbm.at[pl.ds(rowbase, A_CH)])
        return carry

    lax.fori_loop(0, A_NCH, chunk, 0)


@functools.partial(
    pl.kernel,
    out_type=[
        jax.ShapeDtypeStruct((EPR, 128), jnp.float32),
        jax.ShapeDtypeStruct((2 * NP,), jnp.float32),
    ],
    mesh=_mesh(),
    compiler_params=pltpu.CompilerParams(needs_layout_passes=False, use_tc_tiling_on_sc=False),
    scratch_types=[
        pltpu.VMEM((NP,), jnp.float32),
        pltpu.VMEM((A_CH, 128), jnp.int32),
        pltpu.VMEM((A_CH, 128), jnp.float32),
        pltpu.VMEM((A_CH, 128), jnp.float32),
        pltpu.VMEM((16,), jnp.float32),
        pltpu.VMEM((4352,), jnp.float32),
        pltpu.VMEM_SHARED((NP,), jnp.float32),
        pltpu.SemaphoreType.DMA,
    ],
)
def _sc_a2(adst_hbm, dst_hbm, p_hbm, m_hbm, w_hbm, den_hbm,
           tab_v, idx_v, p_v, w_v, m_v, stage_v, den_sp, sem_a):
    c = lax.axis_index("c")
    s = lax.axis_index("s")
    wid = s * 2 + c
    z16 = jnp.zeros((16,), jnp.float32)

    @pl.when(s == 0)
    def _():
        def zero(r, carry):
            stage_v[pl.ds(r * 16, 16)] = z16
            return carry

        lax.fori_loop(0, 4352 // 16, zero, 0)

        def zcp(k, carry):
            pltpu.sync_copy(stage_v, den_sp.at[pl.ds(k * 4352, 4352)])
            return carry

        lax.fori_loop(0, NP // 4352, zcp, 0)

    pltpu.sync_copy(adst_hbm, tab_v)
    pltpu.sync_copy(m_hbm, m_v)
    plsc.subcore_barrier()
    mv = m_v[...]

    def chunk(t, carry):
        rowbase = wid * A_ROWS + t * A_CH
        cp1 = pltpu.async_copy(dst_hbm.at[pl.ds(rowbase, A_CH)], idx_v, sem_a)
        cp2 = pltpu.async_copy(p_hbm.at[pl.ds(rowbase, A_CH)], p_v, sem_a)
        cp1.wait()
        cp2.wait()

        def vec(j, carry2):
            for l in range(8):
                i16 = idx_v[j, pl.ds(l * 16, 16)]
                q = plsc.load_gather(tab_v, [i16])
                z = p_v[j, pl.ds(l * 16, 16)] + q
                lr = jnp.where(z >= 0, z, 0.2 * z)
                w_v[j, pl.ds(l * 16, 16)] = jnp.exp(lr - mv)
            return carry2

        lax.fori_loop(0, A_CH, vec, 0)
        pltpu.sync_copy(w_v, w_hbm.at[pl.ds(rowbase, A_CH)])

        cps = [pltpu.async_copy(w_v.at[j], den_sp.at[idx_v.at[j]],
                                sem_a, add=True)
               for j in range(A_CH)]
        for cp in cps:
            cp.wait()
        return carry

    lax.fori_loop(0, A_NCH, chunk, 0)
    plsc.subcore_barrier()

    @pl.when(s == 0)
    def _():
        def wb(k, carry):
            pltpu.sync_copy(den_sp.at[pl.ds(k * 4352, 4352)], stage_v)
            pltpu.sync_copy(stage_v,
                            den_hbm.at[pl.ds(c * NP + k * 4352, 4352)])
            return carry

        lax.fori_loop(0, NP // 4352, wb, 0)


@functools.partial(
    pl.kernel,
    out_type=jax.ShapeDtypeStruct((2 * NP, H), jnp.float32),
    mesh=_mesh(),
    compiler_params=pltpu.CompilerParams(needs_layout_passes=False, use_tc_tiling_on_sc=False),
    scratch_types=[
        pltpu.VMEM((B_CH, 128), jnp.int32),
        pltpu.VMEM((B_CH, 128), jnp.int32),
        pltpu.VMEM((B_CH, 128), jnp.float32),
        pltpu.VMEM((B_CH * 128, H), jnp.float32),
        pltpu.VMEM((B_CH, 128), jnp.int32),
        pltpu.VMEM((B_CH, 128), jnp.int32),
        pltpu.VMEM((B_CH, 128), jnp.float32),
        pltpu.VMEM((B_CH * 128, H), jnp.float32),
        pltpu.VMEM_SHARED((NP, H), jnp.float32),
        pltpu.SemaphoreType.DMA,
        pltpu.SemaphoreType.DMA,
        pltpu.SemaphoreType.DMA,
        pltpu.SemaphoreType.DMA,
        pltpu.SemaphoreType.DMA,
        pltpu.SemaphoreType.DMA,
    ],
)
def _sc_b(hA_hbm, hB_hbm, src_hbm, dst_hbm, w_hbm, num_hbm,
          s0, d0, w0, r0, s1, d1, w1, r1, num_sp,
          semi0, semi1, semg0, semg1, semc0, semc1):
    c = lax.axis_index("c")
    s = lax.axis_index("s")
    z16 = jnp.zeros((16,), jnp.float32)

    def zero(r, carry):
        r0[r, :] = z16
        return carry

    lax.fori_loop(0, B_STG, zero, 0)

    def zcp(k, carry):
        pltpu.sync_copy(r0.at[pl.ds(0, B_STG)],
                        num_sp.at[pl.ds(s * NPS + k * B_STG, B_STG)])
        return carry

    lax.fori_loop(0, NPS // B_STG, zcp, 0)
    plsc.subcore_barrier()

    def fire_in(t, sv, dv, wv, sem):
        rowbase = s * B_ROWS + t * B_CH
        pltpu.async_copy(src_hbm.at[pl.ds(rowbase, B_CH)], sv, sem)
        pltpu.async_copy(dst_hbm.at[pl.ds(rowbase, B_CH)], dv, sem)
        pltpu.async_copy(w_hbm.at[pl.ds(rowbase, B_CH)], wv, sem)

    def drain_in(sv, dv, wv, sem):
        pltpu.make_async_copy(src_hbm.at[pl.ds(0, B_CH)], sv, sem).wait()
        pltpu.make_async_copy(dst_hbm.at[pl.ds(0, B_CH)], dv, sem).wait()
        pltpu.make_async_copy(w_hbm.at[pl.ds(0, B_CH)], wv, sem).wait()

    def fire_g(sv, rv, sem):
        @pl.when(c == 0)
        def _():
            for j in range(B_CH):
                pltpu.async_copy(hA_hbm.at[sv.at[j]],
                                 rv.at[pl.ds(j * 128, 128)], sem)

        @pl.when(c == 1)
        def _():
            for j in range(B_CH):
                pltpu.async_copy(hB_hbm.at[sv.at[j]],
                                 rv.at[pl.ds(j * 128, 128)], sem)

    def drain_g(rv, sem):
        for j in range(B_CH):
            pltpu.make_async_copy(hA_hbm.at[pl.ds(0, 128)],
                                  rv.at[pl.ds(j * 128, 128)], sem).wait()

    def fire_sc(rv, dv, sem):
        for j in range(B_CH):
            pltpu.async_copy(rv.at[pl.ds(j * 128, 128)],
                             num_sp.at[dv.at[j]], sem, add=True)

    def drain_sc(rv, sem):
        for j in range(B_CH):
            pltpu.make_async_copy(rv.at[pl.ds(j * 128, 128)],
                                  num_sp.at[pl.ds(0, 128)], sem).wait()

    def scale(wv, rv):
        def body(j, carry2):
            base = j * 128
            for l in range(8):
                w16 = wv[j, pl.ds(l * 16, 16)]
                for e in range(16):
                    sp = w16[jnp.full((16,), e, jnp.int32)]
                    r = base + l * 16 + e
                    rv[r, :] = rv[r, :] * sp
            return carry2

        lax.fori_loop(0, B_CH, body, 0)

    G2 = B_NCH // 2
    fire_in(0, s0, d0, w0, semi0)
    drain_in(s0, d0, w0, semi0)
    fire_g(s0, r0, semg0)

    def dchunk(g, carry):
        t0 = 2 * g

        @pl.when(g > 0)
        def _():
            drain_sc(r1, semc1)

        fire_in(t0 + 1, s1, d1, w1, semi1)
        drain_g(r0, semg0)
        scale(w0, r0)
        fire_sc(r0, d0, semc0)
        drain_in(s1, d1, w1, semi1)
        fire_g(s1, r1, semg1)
        drain_sc(r0, semc0)

        @pl.when(g + 1 < G2)
        def _():
            fire_in(t0 + 2, s0, d0, w0, semi0)
            drain_in(s0, d0, w0, semi0)
            fire_g(s0, r0, semg0)

        drain_g(r1, semg1)
        scale(w1, r1)
        fire_sc(r1, d1, semc1)
        return carry

    lax.fori_loop(0, G2, dchunk, 0)
    drain_sc(r1, semc1)
    plsc.subcore_barrier()
    def wb(k, carry):
        off = s * NPS + k * B_STG
        pltpu.sync_copy(num_sp.at[pl.ds(off, B_STG)], r0.at[pl.ds(0, B_STG)])
        pltpu.sync_copy(r0.at[pl.ds(0, B_STG)],
                        num_hbm.at[pl.ds(c * NP + off, B_STG)])
        return carry

    lax.fori_loop(0, NPS // B_STG, wb, 0)


def _tc2_body(hA_ref, hB_ref, as_ref, ad_ref, d0_ref, d1_ref, n0_ref, n1_ref,
              ms_ref, md_ref, b_ref, o_ref):
    a_s = as_ref[...]
    a_d = ad_ref[...]
    M = jnp.maximum(ms_ref[0, 0] + md_ref[0, 0], 0.0)
    z = a_s + a_d
    lr = jnp.where(z >= 0, z, 0.2 * z)
    wself = jnp.exp(lr - M)
    den = d0_ref[...] + d1_ref[...] + wself + 1e-16
    nA = (n0_ref[...] + wself * hA_ref[...]) / den
    nB = (n1_ref[...] + wself * hB_ref[...]) / den
    o_ref[...] = jnp.concatenate([nA, nB], axis=1) + b_ref[0:1, :]


def _tc2(hA, hB, as_col, ad_col, d0, d1, n0, n1, ms, md, bias8):
    col = pl.BlockSpec((BN, 1), lambda i: (i, 0))
    half = pl.BlockSpec((BN, H), lambda i: (i, 0))
    return pl.pallas_call(
        _tc2_body,
        grid=(G1,),
        in_specs=[
            half, half, col, col, col, col, half, half,
            pl.BlockSpec((1, 1), lambda i: (0, 0), memory_space=pltpu.SMEM),
            pl.BlockSpec((1, 1), lambda i: (0, 0), memory_space=pltpu.SMEM),
            pl.BlockSpec((8, D), lambda i: (0, 0)),
        ],
        out_specs=pl.BlockSpec((BN, D), lambda i: (i, 0)),
        out_shape=jax.ShapeDtypeStruct((NP, D), jnp.float32),
    )(hA, hB, as_col, ad_col, d0, d1, n0, n1, ms, md, bias8)


def kernel(x, edge_index, W, att_src, att_dst, bias):
    src = edge_index[0].astype(jnp.int32)
    dst = edge_index[1].astype(jnp.int32)

    xp = jnp.concatenate([x.astype(jnp.float32),
                          jnp.zeros((NP - N, D), jnp.float32)], axis=0)
    sel8 = jnp.concatenate([att_src[None], att_dst[None],
                            jnp.zeros((6, D), jnp.float32)], axis=0)
    bias8 = jnp.broadcast_to(bias[None, :], (8, D)).astype(jnp.float32)

    hA, hB, asH, adH, ms, md = _tc1(xp, W.astype(jnp.float32), sel8)

    pad = EP - E
    srcP = jnp.concatenate([src, jnp.zeros((pad,), jnp.int32)]).reshape(EPR, 128)
    dstP = jnp.concatenate([dst, jnp.full((pad,), N, jnp.int32)]).reshape(EPR, 128)

    M16 = jnp.broadcast_to(jnp.maximum(ms[0, 0] + md[0, 0], 0.0), (16,))

    p_edge = _sc_a1(asH.reshape(NP), srcP)
    w_edge, den = _sc_a2(adH.reshape(NP), dstP, p_edge, M16)
    num = _sc_b(hA, hB, srcP, dstP, w_edge)

    as_col = asH.reshape(NP, 1)
    ad_col = adH.reshape(NP, 1)
    d0 = den[:NP].reshape(NP, 1)
    d1 = den[NP:].reshape(NP, 1)
    n0 = num[:NP]
    n1 = num[NP:]

    out = _tc2(hA, hB, as_col, ad_col, d0, d1, n0, n1, ms, md, bias8)
    return out[:N]
